# Initial kernel scaffold; baseline (speedup 1.0000x reference)
import jax
import jax.numpy as jnp
from jax.experimental import pallas as pl


def kernel(x, edge_index, edge_weight, W, b):
    raise NotImplementedError("write your pallas kernel here")


# speedup vs baseline: 4.5421x; 4.5421x over previous
import functools

import jax
import jax.numpy as jnp
from jax import lax
from jax.experimental import pallas as pl
from jax.experimental.pallas import tpu as pltpu
from jax.experimental.pallas import tpu_sc as plsc

N = 10000
NP = 10240
D = 128
E = 320000
NC = 2
NS = 16
NW = NC * NS
EPT = E // NW
C = 80
NCHUNK = EPT // C
ROWS_PT = NP // NS
ZR = 128
LG = D // 16


def _sc_agg_body(x_hbm, src_hbm, dst_hbm, ew_hbm, out_hbm,
                 src_v, dst_v, w_v, rows_v, zbuf, acc, sem):
    c = lax.axis_index("c")
    s = lax.axis_index("s")
    wid = s * NC + c

    zeros16 = jnp.zeros((16,), jnp.float32)

    def zfill(i, _):
        for j in range(LG):
            zbuf[i, pl.ds(j * 16, 16)] = zeros16
        return 0

    lax.fori_loop(0, ZR, zfill, 0)
    for k in range(ROWS_PT // ZR):
        pltpu.sync_copy(zbuf, acc.at[pl.ds(s * ROWS_PT + k * ZR, ZR)])
    plsc.subcore_barrier()

    def chunk(i, _):
        base = wid * EPT + i * C
        pltpu.sync_copy(src_hbm.at[pl.ds(base, C)], src_v)
        pltpu.sync_copy(dst_hbm.at[pl.ds(base, C)], dst_v)
        pltpu.sync_copy(ew_hbm.at[pl.ds(base, C)], w_v)
        pltpu.async_copy(x_hbm.at[src_v], rows_v, sem).wait()

        def scale(g, _):
            w16 = w_v[pl.ds(g * 16, 16)]
            for e in range(16):
                w_e = w16[e]
                row = g * 16 + e
                for j in range(LG):
                    rows_v[row, pl.ds(j * 16, 16)] = (
                        rows_v[row, pl.ds(j * 16, 16)] * w_e
                    )
            return 0

        lax.fori_loop(0, C // 16, scale, 0)
        pltpu.sync_copy(rows_v, acc.at[dst_v], add=True)
        return 0

    lax.fori_loop(0, NCHUNK, chunk, 0)
    plsc.subcore_barrier()

    pltpu.sync_copy(acc.at[pl.ds(s * ROWS_PT, ROWS_PT)],
                    out_hbm.at[c, pl.ds(s * ROWS_PT, ROWS_PT)])


_sc_agg = functools.partial(
    pl.kernel,
    mesh=plsc.VectorSubcoreMesh(core_axis_name="c", subcore_axis_name="s"),
    out_type=jax.ShapeDtypeStruct((NC, NP, D), jnp.float32),
    scratch_types=[
        pltpu.VMEM((C,), jnp.int32),
        pltpu.VMEM((C,), jnp.int32),
        pltpu.VMEM((C,), jnp.float32),
        pltpu.VMEM((C, D), jnp.float32),
        pltpu.VMEM((ZR, D), jnp.float32),
        pltpu.VMEM_SHARED((NP, D), jnp.float32),
        pltpu.SemaphoreType.DMA,
    ],
)(_sc_agg_body)


def _tc_body(p0_ref, p1_ref, w_ref, b_ref, o_ref):
    acc = p0_ref[0] + p1_ref[0]
    o_ref[...] = (
        jnp.dot(acc, w_ref[...], preferred_element_type=jnp.float32)
        + b_ref[...]
    )


BN = 1000


def kernel(x, edge_index, edge_weight, W, b):
    src = edge_index[0]
    dst = edge_index[1]
    partials = _sc_agg(x, src, dst, edge_weight)
    out = pl.pallas_call(
        _tc_body,
        grid=(N // BN,),
        in_specs=[
            pl.BlockSpec((1, BN, D), lambda i: (0, i, 0)),
            pl.BlockSpec((1, BN, D), lambda i: (1, i, 0)),
            pl.BlockSpec((D, D), lambda i: (0, 0)),
            pl.BlockSpec((1, D), lambda i: (0, 0)),
        ],
        out_specs=pl.BlockSpec((BN, D), lambda i: (i, 0)),
        out_shape=jax.ShapeDtypeStruct((N, D), jnp.float32),
    )(partials, partials, W, b.reshape(1, D))
    return out


# speedup vs baseline: 12.3667x; 2.7227x over previous
import functools

import jax
import jax.numpy as jnp
from jax import lax
from jax.experimental import pallas as pl
from jax.experimental.pallas import tpu as pltpu
from jax.experimental.pallas import tpu_sc as plsc

N = 10000
NP = 10240
D = 128
E = 320000
NC = 2
NS = 16
NW = NC * NS
EPT = E // NW
C = 80
NCHUNK = EPT // C
ROWS_PT = NP // NS
LG = D // 16
CG = C // 16
SHIFT = 14


def _sc_agg_body(x_hbm, sd_hbm, ew_hbm, out_hbm,
                 sd_a, sidx0, sidx1, sidx2, didx0, didx1, didx2,
                 wb0, wb1, wb2, rows0, rows1, rows2, acc,
                 gs0, gs1, gs2, ss0, ss1, ss2):
    c = lax.axis_index("c")
    s = lax.axis_index("s")
    wid = s * NC + c
    rows = (rows0, rows1, rows2)
    sidx = (sidx0, sidx1, sidx2)
    didx = (didx0, didx1, didx2)
    wb = (wb0, wb1, wb2)
    gsem = (gs0, gs1, gs2)
    ssem = (ss0, ss1, ss2)

    pltpu.sync_copy(sd_hbm.at[wid], sd_a)

    zeros16 = jnp.zeros((16,), jnp.float32)

    def zfill(i, _):
        for j in range(LG):
            rows0[i, pl.ds(j * 16, 16)] = zeros16
        return 0

    lax.fori_loop(0, C, zfill, 0)
    for k in range(ROWS_PT // C):
        pltpu.sync_copy(rows0, acc.at[pl.ds(s * ROWS_PT + k * C, C)])
    plsc.subcore_barrier()

    def unpack(j, b):
        sb, db = sidx[b], didx[b]

        def grp(g, _):
            sd = sd_a[j, pl.ds(g * 16, 16)]
            sb[pl.ds(g * 16, 16)] = lax.shift_right_logical(sd, SHIFT)
            db[pl.ds(g * 16, 16)] = lax.bitwise_and(sd, (1 << SHIFT) - 1)
            return 0

        lax.fori_loop(0, CG, grp, 0)

    def fetch_start(j, b):
        pltpu.make_async_copy(x_hbm.at[sidx[b]], rows[b], gsem[b]).start()
        pltpu.make_async_copy(
            ew_hbm.at[pl.ds(wid * EPT + j * C, C)], wb[b], gsem[b]).start()

    def fetch_wait(j, b):
        pltpu.make_async_copy(x_hbm.at[sidx[b]], rows[b], gsem[b]).wait()
        pltpu.make_async_copy(
            ew_hbm.at[pl.ds(wid * EPT + j * C, C)], wb[b], gsem[b]).wait()

    def scatter_start(j, b):
        pltpu.make_async_copy(rows[b], acc.at[didx[b]], ssem[b]).start(
            add=True)

    def scatter_wait(j, b):
        pltpu.make_async_copy(rows[b], acc.at[didx[b]], ssem[b]).wait()

    def scale(j, b):
        r, w = rows[b], wb[b]

        def grp(g, _):
            w16 = w[pl.ds(g * 16, 16)]
            for e in range(16):
                w_e = w16[e]
                row = g * 16 + e
                for l in range(LG):
                    r[row, pl.ds(l * 16, 16)] = r[row, pl.ds(l * 16, 16)] * w_e
            return 0

        lax.fori_loop(0, CG, grp, 0)

    unpack(0, 0)
    fetch_start(0, 0)
    unpack(1, 1)
    fetch_start(1, 1)

    fetch_wait(0, 0)
    scale(0, 0)
    scatter_start(0, 0)
    unpack(2, 2)
    fetch_start(2, 2)

    def slots(it, _):
        j0 = 1 + it * 3
        for k in range(3):
            j = j0 + k
            b = (1 + k) % 3
            bprev = k % 3
            fetch_wait(j, b)
            scale(j, b)
            scatter_start(j, b)

            @pl.when(j + 2 < NCHUNK)
            def _refill():
                scatter_wait(j - 1, bprev)
                unpack(j + 2, bprev)
                fetch_start(j + 2, bprev)

        return 0

    lax.fori_loop(0, (NCHUNK - 2) // 3, slots, 0)

    jlast = NCHUNK - 1
    blast = jlast % 3
    fetch_wait(jlast, blast)
    scale(jlast, blast)
    scatter_start(jlast, blast)
    scatter_wait(jlast - 2, (jlast - 2) % 3)
    scatter_wait(jlast - 1, (jlast - 1) % 3)
    scatter_wait(jlast, blast)
    plsc.subcore_barrier()

    pltpu.sync_copy(acc.at[pl.ds(s * ROWS_PT, ROWS_PT)],
                    out_hbm.at[c, pl.ds(s * ROWS_PT, ROWS_PT)])


_sc_agg = functools.partial(
    pl.kernel,
    mesh=plsc.VectorSubcoreMesh(core_axis_name="c", subcore_axis_name="s"),
    out_type=jax.ShapeDtypeStruct((NC, NP, D), jnp.float32),
    scratch_types=[
        pltpu.VMEM((NCHUNK, C), jnp.int32),
        pltpu.VMEM((C,), jnp.int32),
        pltpu.VMEM((C,), jnp.int32),
        pltpu.VMEM((C,), jnp.int32),
        pltpu.VMEM((C,), jnp.int32),
        pltpu.VMEM((C,), jnp.int32),
        pltpu.VMEM((C,), jnp.int32),
        pltpu.VMEM((C,), jnp.float32),
        pltpu.VMEM((C,), jnp.float32),
        pltpu.VMEM((C,), jnp.float32),
        pltpu.VMEM((C, D), jnp.float32),
        pltpu.VMEM((C, D), jnp.float32),
        pltpu.VMEM((C, D), jnp.float32),
        pltpu.VMEM_SHARED((NP, D), jnp.float32),
        pltpu.SemaphoreType.DMA,
        pltpu.SemaphoreType.DMA,
        pltpu.SemaphoreType.DMA,
        pltpu.SemaphoreType.DMA,
        pltpu.SemaphoreType.DMA,
        pltpu.SemaphoreType.DMA,
    ],
)(_sc_agg_body)


def _tc_body(p0_ref, p1_ref, w_ref, b_ref, o_ref):
    acc = p0_ref[0] + p1_ref[0]
    o_ref[...] = (
        jnp.dot(acc, w_ref[...], preferred_element_type=jnp.float32)
        + b_ref[...]
    )


BN = 1000


def kernel(x, edge_index, edge_weight, W, b):
    packed = (edge_index[0] << SHIFT) | edge_index[1]
    sd = packed.reshape(NW, NCHUNK, C)
    partials = _sc_agg(x, sd, edge_weight)
    out = pl.pallas_call(
        _tc_body,
        grid=(N // BN,),
        in_specs=[
            pl.BlockSpec((1, BN, D), lambda i: (0, i, 0)),
            pl.BlockSpec((1, BN, D), lambda i: (1, i, 0)),
            pl.BlockSpec((D, D), lambda i: (0, 0)),
            pl.BlockSpec((1, D), lambda i: (0, 0)),
        ],
        out_specs=pl.BlockSpec((BN, D), lambda i: (i, 0)),
        out_shape=jax.ShapeDtypeStruct((N, D), jnp.float32),
    )(partials, partials, W, b.reshape(1, D))
    return out
